# baseline (speedup 1.0000x reference)
import jax
import jax.numpy as jnp
from jax.experimental import pallas as pl
from jax.experimental.pallas import tpu as pltpu

_NEG = -1e9
_MASKED = -1e30
_SRC_BLK = 8


def _net_kernel(sstar_ref, x_ref, adj_ref, adjt_ref, hints_ref, vecs_ref,
                wcat_ref, wo2_ref, wdec_ref, out_ref, awe_ref):
    b = pl.program_id(0)
    pair = x_ref.shape[0]
    n = adj_ref.shape[1]
    t_total = hints_ref.shape[2]
    h = wo2_ref.shape[1]
    nsteps = max(1, t_total - 1)

    vecs = vecs_ref[:, :]
    win = vecs[0:1]
    whint = vecs[1:2]
    we = vecs[2:3]
    wcat = wcat_ref[:, :]
    wo2 = wo2_ref[:, :]
    wdec = wdec_ref[:, :]

    for p in range(pair):
        adjb = adj_ref[p]
        for s0 in range(0, n, _SRC_BLK):
            a = adjb[s0:s0 + _SRC_BLK][:, :, None]
            awe_ref[p, s0:s0 + _SRC_BLK] = jnp.where(a > 0.5, a * we[0],
                                                     _MASKED)

    empties = [jnp.max(adjt_ref[p], axis=1, keepdims=True) <= 0.5
               for p in range(pair)]
    lane_t = jax.lax.broadcasted_iota(jnp.int32, (n, t_total), 1)

    def one_batch(p, i, hidden):
        x = x_ref[p]
        hb = hints_ref[p]
        hc = jnp.sum(jnp.where(lane_t == i, hb, 0.0), axis=1,
                     keepdims=True)
        enc = x * win + hc * whint
        z = jnp.concatenate([enc, hidden], axis=1)
        mm = jnp.dot(z, wcat, preferred_element_type=jnp.float32)
        m1 = mm[:, :h]
        m2 = mm[:, h:2 * h]
        zo = mm[:, 2 * h:]

        m = jnp.full((n, h), _MASKED, dtype=jnp.float32)
        for s0 in range(0, n, _SRC_BLK):
            m1b = m1[s0:s0 + _SRC_BLK]
            tblk = (m1b[:, None, :] + m2[None, :, :]) + awe_ref[p, s0:s0 + _SRC_BLK]
            m = jnp.maximum(m, jnp.max(tblk, axis=0))
        msgs = jnp.where(empties[p], _NEG, jnp.maximum(m, 0.0))
        h_new = jnp.maximum(zo + jnp.dot(msgs, wo2,
                                         preferred_element_type=jnp.float32),
                            0.0)
        out_cand = jnp.dot(h_new, wdec,
                           preferred_element_type=jnp.float32)

        @pl.when(i == sstar_ref[pair * b + p])
        def _():
            out_ref[p] = out_cand

        return h_new

    def step(i, hiddens):
        return tuple(one_batch(p, i, hiddens[p]) for p in range(pair))

    jax.lax.fori_loop(
        0, nsteps, step,
        tuple(jnp.zeros((n, h), jnp.float32) for _ in range(pair)))


def kernel(node_inputs, adj, hints, lengths, W_enc_in, W_enc_hint, W_edge,
           W_m1, W_m2, W_o1, W_o2, W_dec_out, W_dec_hint):
    B, N, _ = node_inputs.shape
    T = hints.shape[0]
    H = W_o1.shape[1]

    hints_nt = jnp.transpose(hints, (1, 2, 0))
    adj_t = jnp.swapaxes(adj, 1, 2)
    vecs = jnp.concatenate([W_enc_in, W_enc_hint, W_edge], axis=0)
    wcat = jnp.concatenate([W_m1, W_m2, W_o1], axis=1)
    sstar = jnp.clip(lengths - 2, 0, max(0, T - 2)).astype(jnp.int32)

    PAIR = 2
    return pl.pallas_call(
        _net_kernel,
        grid=(B // PAIR,),
        in_specs=[
            pl.BlockSpec(memory_space=pltpu.SMEM),
            pl.BlockSpec((PAIR, N, 1), lambda b: (b, 0, 0)),
            pl.BlockSpec((PAIR, N, N), lambda b: (b, 0, 0)),
            pl.BlockSpec((PAIR, N, N), lambda b: (b, 0, 0)),
            pl.BlockSpec((PAIR, N, T), lambda b: (b, 0, 0)),
            pl.BlockSpec((3, H), lambda b: (0, 0)),
            pl.BlockSpec((2 * H, 3 * H), lambda b: (0, 0)),
            pl.BlockSpec((H, H), lambda b: (0, 0)),
            pl.BlockSpec((H, 1), lambda b: (0, 0)),
        ],
        out_specs=pl.BlockSpec((PAIR, N, 1), lambda b: (b, 0, 0)),
        out_shape=jax.ShapeDtypeStruct((B, N, 1), jnp.float32),
        scratch_shapes=[pltpu.VMEM((PAIR, N, N, H), jnp.float32)],
        compiler_params=pltpu.CompilerParams(
            dimension_semantics=("parallel",)),
    )(sstar, node_inputs, adj, adj_t, hints_nt, vecs, wcat,
      W_o2, W_dec_out)[:, :, 0]


# speedup vs baseline: 1.0066x; 1.0066x over previous
import jax
import jax.numpy as jnp
from jax.experimental import pallas as pl
from jax.experimental.pallas import tpu as pltpu

_NEG = -1e9
_MASKED = -1e30
_SRC_BLK = 8


def _net_kernel(sstar_ref, x_ref, adj_ref, adjt_ref, hints_ref, vecs_ref,
                wcat_ref, wo2_ref, wdec_ref, out_ref, awe_ref, mm_ref):
    b = pl.program_id(0)
    pair = x_ref.shape[0]
    n = adj_ref.shape[1]
    t_total = hints_ref.shape[2]
    h = wo2_ref.shape[1]
    nsteps = max(1, t_total - 1)

    vecs = vecs_ref[:, :]
    win = vecs[0:1]
    whint = vecs[1:2]
    we = vecs[2:3]
    wcat = wcat_ref[:, :]
    wo2 = wo2_ref[:, :]
    wdec = wdec_ref[:, :]

    for p in range(pair):
        adjb = adj_ref[p]
        for s0 in range(0, n, _SRC_BLK):
            a = adjb[s0:s0 + _SRC_BLK][:, :, None]
            awe_ref[p, s0:s0 + _SRC_BLK] = jnp.where(a > 0.5, a * we[0],
                                                     _MASKED)

    empties = [jnp.max(adjt_ref[p], axis=1, keepdims=True) <= 0.5
               for p in range(pair)]
    lane_t = jax.lax.broadcasted_iota(jnp.int32, (n, t_total), 1)

    def encoder(p, i):
        hc = jnp.sum(jnp.where(lane_t == i, hints_ref[p], 0.0), axis=1,
                     keepdims=True)
        return x_ref[p] * win + hc * whint

    def one_batch(p, i, hidden, enc):
        z = jnp.concatenate([enc, hidden], axis=1)
        mm_ref[p] = jnp.dot(z, wcat, preferred_element_type=jnp.float32)
        m2 = mm_ref[p, :, h:2 * h]

        m = jnp.full((n, h), _MASKED, dtype=jnp.float32)
        for s0 in range(0, n, _SRC_BLK):
            m1b = mm_ref[p, s0:s0 + _SRC_BLK, :h]
            tblk = (m1b[:, None, :] + m2[None, :, :]) + awe_ref[p, s0:s0 + _SRC_BLK]
            m = jnp.maximum(m, jnp.max(tblk, axis=0))
        msgs = jnp.where(empties[p], _NEG, jnp.maximum(m, 0.0))
        h_new = jnp.maximum(mm_ref[p, :, 2 * h:] +
                            jnp.dot(msgs, wo2,
                                    preferred_element_type=jnp.float32),
                            0.0)

        @pl.when(i == sstar_ref[pair * b + p])
        def _():
            out_ref[p] = jnp.dot(h_new, wdec,
                                 preferred_element_type=jnp.float32)

        return h_new, encoder(p, i + 1)

    def step(i, carry):
        return tuple(one_batch(p, i, *carry[p]) for p in range(pair))

    jax.lax.fori_loop(
        0, nsteps, step,
        tuple((jnp.zeros((n, h), jnp.float32), encoder(p, 0))
              for p in range(pair)))


def kernel(node_inputs, adj, hints, lengths, W_enc_in, W_enc_hint, W_edge,
           W_m1, W_m2, W_o1, W_o2, W_dec_out, W_dec_hint):
    B, N, _ = node_inputs.shape
    T = hints.shape[0]
    H = W_o1.shape[1]

    hints_nt = jnp.transpose(hints, (1, 2, 0))
    adj_t = jnp.swapaxes(adj, 1, 2)
    vecs = jnp.concatenate([W_enc_in, W_enc_hint, W_edge], axis=0)
    wcat = jnp.concatenate([W_m1, W_m2, W_o1], axis=1)
    sstar = jnp.clip(lengths - 2, 0, max(0, T - 2)).astype(jnp.int32)

    PAIR = 1
    return pl.pallas_call(
        _net_kernel,
        grid=(B // PAIR,),
        in_specs=[
            pl.BlockSpec(memory_space=pltpu.SMEM),
            pl.BlockSpec((PAIR, N, 1), lambda b: (b, 0, 0)),
            pl.BlockSpec((PAIR, N, N), lambda b: (b, 0, 0)),
            pl.BlockSpec((PAIR, N, N), lambda b: (b, 0, 0)),
            pl.BlockSpec((PAIR, N, T), lambda b: (b, 0, 0)),
            pl.BlockSpec((3, H), lambda b: (0, 0)),
            pl.BlockSpec((2 * H, 3 * H), lambda b: (0, 0)),
            pl.BlockSpec((H, H), lambda b: (0, 0)),
            pl.BlockSpec((H, 1), lambda b: (0, 0)),
        ],
        out_specs=pl.BlockSpec((PAIR, N, 1), lambda b: (b, 0, 0)),
        out_shape=jax.ShapeDtypeStruct((B, N, 1), jnp.float32),
        scratch_shapes=[pltpu.VMEM((PAIR, N, N, H), jnp.float32),
                        pltpu.VMEM((PAIR, N, 3 * H), jnp.float32)],
        compiler_params=pltpu.CompilerParams(
            dimension_semantics=("parallel",)),
    )(sstar, node_inputs, adj, adj_t, hints_nt, vecs, wcat,
      W_o2, W_dec_out)[:, :, 0]
